# baseline (speedup 1.0000x reference)
import functools

import jax
import jax.numpy as jnp
from jax import lax
from jax.experimental import pallas as pl
from jax.experimental.pallas import tpu as pltpu
from jax.experimental.pallas import tpu_sc as plsc

NUM_TOKENS = 8192
D_IN = 2048
D_OUT = 2048
RANK = 16
NUM_EXPERTS = 16
SCALING = 32 / float(RANK)
ER = NUM_EXPERTS * RANK

TOKEN_BLOCK = 512

NC, NS = 2, 16
NW = NC * NS
TPW = NUM_TOKENS // NW
HALF = TPW // 2


def _inter_kernel(x_ref, a_ref, o_ref):
    o_ref[...] = jax.lax.dot_general(
        x_ref[...], a_ref[...], (((1,), (1,)), ((), ())),
        preferred_element_type=jnp.float32)


QTR = TPW // 4


def _sc_mask_body(inter_hbm, eid_hbm, out_hbm,
                  eid_v, in0, in1, out_v, sem0, sem1):
    wid = lax.axis_index("s") * NC + lax.axis_index("c")
    base = wid * TPW

    pltpu.sync_copy(eid_hbm.at[pl.ds(base, TPW)], eid_v)

    ins = (in0, in1)
    sems = (sem0, sem1)

    def fetch(q, buf):
        return pltpu.async_copy(
            inter_hbm.at[pl.ds(base + q * QTR, QTR), :],
            ins[buf], sems[buf])

    cps = [fetch(0, 0), fetch(1, 1)]

    zero16 = jnp.zeros((16,), jnp.float32)

    def zr(r, carry):
        for k in range(ER // 16):
            out_v[r, pl.ds(k * 16, 16)] = zero16
        return carry

    lax.fori_loop(0, TPW, zr, None)

    for q in range(4):
        buf = q % 2
        cps[buf].wait()
        in_v = ins[buf]

        def put(g, carry, q=q, in_v=in_v):
            e16 = eid_v[pl.ds(q * QTR + g * 16, 16)]
            for j in range(16):
                t = g * 16 + j
                off = e16[j] * RANK
                out_v[q * QTR + t, pl.ds(off, RANK)] = (
                    in_v[t, pl.ds(off, RANK)])
            return carry

        lax.fori_loop(0, QTR // 16, put, None)
        if q + 2 < 4:
            cps[buf] = fetch(q + 2, buf)

    pltpu.sync_copy(out_v, out_hbm.at[pl.ds(base, TPW), :])


def _sc_mask(inter_all, eids):
    return pl.kernel(
        _sc_mask_body,
        out_type=jax.ShapeDtypeStruct((NUM_TOKENS, ER), jnp.float32),
        mesh=plsc.VectorSubcoreMesh(core_axis_name="c", subcore_axis_name="s"),
        scratch_types=[
            pltpu.VMEM((TPW,), jnp.int32),
            pltpu.VMEM((QTR, ER), jnp.float32),
            pltpu.VMEM((QTR, ER), jnp.float32),
            pltpu.VMEM((TPW, ER), jnp.float32),
            pltpu.SemaphoreType.DMA,
            pltpu.SemaphoreType.DMA,
        ],
    )(inter_all, eids)


def _final_kernel(x_ref, m_ref, w_ref, b_ref, bt_ref, o_ref):
    base = jax.lax.dot_general(
        x_ref[...], w_ref[...], (((1,), (1,)), ((), ())),
        preferred_element_type=jnp.float32)
    delta = jax.lax.dot_general(
        m_ref[...], bt_ref[...], (((1,), (0,)), ((), ())),
        preferred_element_type=jnp.float32)
    o_ref[...] = base + b_ref[...] + SCALING * delta


@functools.partial(jax.jit, static_argnames=())
def kernel(x, expert_ids, W, b, lora_a, lora_b):
    n_tokens = x.shape[0]
    eids = expert_ids.astype(jnp.int32)
    a_flat = lora_a.reshape(ER, D_IN)
    bt = lora_b.transpose(0, 2, 1).reshape(ER, D_OUT)
    b2 = b.reshape(1, D_OUT)

    inter_all = pl.pallas_call(
        _inter_kernel,
        grid=(n_tokens // 512,),
        in_specs=[
            pl.BlockSpec((512, D_IN), lambda i: (i, 0)),
            pl.BlockSpec((ER, D_IN), lambda i: (0, 0)),
        ],
        out_specs=pl.BlockSpec((512, ER), lambda i: (i, 0)),
        out_shape=jax.ShapeDtypeStruct((n_tokens, ER), jnp.float32),
    )(x, a_flat)

    masked = _sc_mask(inter_all, eids)

    out = pl.pallas_call(
        _final_kernel,
        grid=(n_tokens // TOKEN_BLOCK,),
        in_specs=[
            pl.BlockSpec((TOKEN_BLOCK, D_IN), lambda i: (i, 0)),
            pl.BlockSpec((TOKEN_BLOCK, ER), lambda i: (i, 0)),
            pl.BlockSpec((D_OUT, D_IN), lambda i: (0, 0)),
            pl.BlockSpec((1, D_OUT), lambda i: (0, 0)),
            pl.BlockSpec((ER, D_OUT), lambda i: (0, 0)),
        ],
        out_specs=pl.BlockSpec((TOKEN_BLOCK, D_OUT), lambda i: (i, 0)),
        out_shape=jax.ShapeDtypeStruct((n_tokens, D_OUT), jnp.float32),
    )(x, masked, W, b2, bt)
    return out


# speedup vs baseline: 1.0249x; 1.0249x over previous
import functools

import jax
import jax.numpy as jnp
from jax import lax
from jax.experimental import pallas as pl
from jax.experimental.pallas import tpu as pltpu
from jax.experimental.pallas import tpu_sc as plsc

NUM_TOKENS = 8192
D_IN = 2048
D_OUT = 2048
RANK = 16
NUM_EXPERTS = 16
SCALING = 32 / float(RANK)
ER = NUM_EXPERTS * RANK

TOKEN_BLOCK = 512

NC, NS = 2, 16
NW = NC * NS
TPW = NUM_TOKENS // NW
HALF = TPW // 2


def _inter_kernel(x_ref, a_ref, o_ref):
    o_ref[...] = jax.lax.dot_general(
        x_ref[...], a_ref[...], (((1,), (1,)), ((), ())),
        preferred_element_type=jnp.float32)


QTR = TPW // 4


def _sc_mask_body(inter_hbm, eid_hbm, out_hbm,
                  eid_v, in0, in1, out_v, sem0, sem1):
    wid = lax.axis_index("s") * NC + lax.axis_index("c")
    base = wid * TPW

    pltpu.sync_copy(eid_hbm.at[pl.ds(base, TPW)], eid_v)

    ins = (in0, in1)
    sems = (sem0, sem1)

    def fetch(q, buf):
        return pltpu.async_copy(
            inter_hbm.at[pl.ds(base + q * QTR, QTR), :],
            ins[buf], sems[buf])

    cps = [fetch(0, 0), fetch(1, 1)]

    zero16 = jnp.zeros((16,), jnp.float32)

    def zr(r, carry):
        for k in range(ER // 16):
            out_v[r, pl.ds(k * 16, 16)] = zero16
        return carry

    lax.fori_loop(0, TPW, zr, None)

    for q in range(4):
        buf = q % 2
        cps[buf].wait()
        in_v = ins[buf]

        def put(g, carry, q=q, in_v=in_v):
            e16 = eid_v[pl.ds(q * QTR + g * 16, 16)]
            for j in range(16):
                t = g * 16 + j
                off = e16[j] * RANK
                out_v[q * QTR + t, pl.ds(off, RANK)] = (
                    in_v[t, pl.ds(off, RANK)])
            return carry

        lax.fori_loop(0, QTR // 16, put, None)
        if q + 2 < 4:
            cps[buf] = fetch(q + 2, buf)

    pltpu.sync_copy(out_v, out_hbm.at[pl.ds(base, TPW), :])


def _sc_mask(inter_all, eids):
    return pl.kernel(
        _sc_mask_body,
        out_type=jax.ShapeDtypeStruct((NUM_TOKENS, ER), jnp.float32),
        mesh=plsc.VectorSubcoreMesh(core_axis_name="c", subcore_axis_name="s"),
        scratch_types=[
            pltpu.VMEM((TPW,), jnp.int32),
            pltpu.VMEM((QTR, ER), jnp.float32),
            pltpu.VMEM((QTR, ER), jnp.float32),
            pltpu.VMEM((TPW, ER), jnp.float32),
            pltpu.SemaphoreType.DMA,
            pltpu.SemaphoreType.DMA,
        ],
    )(inter_all, eids)


def _final_kernel(x_ref, m_ref, w_ref, b_ref, bt_ref, o_ref):
    base = jax.lax.dot_general(
        x_ref[...], w_ref[...], (((1,), (1,)), ((), ())),
        preferred_element_type=jnp.float32)
    delta = jax.lax.dot_general(
        m_ref[...], bt_ref[...], (((1,), (0,)), ((), ())),
        preferred_element_type=jnp.float32)
    o_ref[...] = base + b_ref[...] + SCALING * delta


@functools.partial(jax.jit, static_argnames=())
def kernel(x, expert_ids, W, b, lora_a, lora_b):
    n_tokens = x.shape[0]
    eids = expert_ids.astype(jnp.int32)
    a_flat = lora_a.reshape(ER, D_IN)
    bt = lora_b.transpose(0, 2, 1).reshape(ER, D_OUT)
    b2 = b.reshape(1, D_OUT)

    inter_all = pl.pallas_call(
        _inter_kernel,
        grid=(n_tokens // 1024,),
        in_specs=[
            pl.BlockSpec((1024, D_IN), lambda i: (i, 0)),
            pl.BlockSpec((ER, D_IN), lambda i: (0, 0)),
        ],
        out_specs=pl.BlockSpec((1024, ER), lambda i: (i, 0)),
        out_shape=jax.ShapeDtypeStruct((n_tokens, ER), jnp.float32),
    )(x, a_flat)

    masked = _sc_mask(inter_all, eids)

    out = pl.pallas_call(
        _final_kernel,
        grid=(n_tokens // TOKEN_BLOCK,),
        in_specs=[
            pl.BlockSpec((TOKEN_BLOCK, D_IN), lambda i: (i, 0)),
            pl.BlockSpec((TOKEN_BLOCK, ER), lambda i: (i, 0)),
            pl.BlockSpec((D_OUT, D_IN), lambda i: (0, 0)),
            pl.BlockSpec((1, D_OUT), lambda i: (0, 0)),
            pl.BlockSpec((ER, D_OUT), lambda i: (0, 0)),
        ],
        out_specs=pl.BlockSpec((TOKEN_BLOCK, D_OUT), lambda i: (i, 0)),
        out_shape=jax.ShapeDtypeStruct((n_tokens, D_OUT), jnp.float32),
    )(x, masked, W, b2, bt)
    return out
